# baseline (device time: 30821 ns/iter reference)
import functools

import jax
import jax.numpy as jnp
from jax import lax
from jax.experimental import pallas as pl
from jax.experimental.pallas import tpu as pltpu

N_DEV = 8
K = 16
N_ROUNDS = 3
NEG_INF = float("-inf")


def _topk_iter(x, k):
    m = jnp.max(x, axis=1, keepdims=True)
    cols = [m]
    for _ in range(k - 1):
        m = jnp.max(jnp.where(x < m, x, NEG_INF), axis=1, keepdims=True)
        cols.append(m)
    return jnp.concatenate(cols, axis=1)


def kernel(x):
    m, n = x.shape

    def body(x_ref, out_ref, gather_ref, send_sems, recv_sems):
        my_id = lax.axis_index("i")
        partners = [my_id ^ (1 << r) for r in range(N_ROUNDS)]

        barrier = pltpu.get_barrier_semaphore()
        for p in partners:
            pl.semaphore_signal(
                barrier, inc=1,
                device_id=(p,), device_id_type=pl.DeviceIdType.MESH,
            )
        pl.semaphore_wait(barrier, N_ROUNDS)

        gather_ref[0] = _topk_iter(x_ref[:, :], K).T

        for r in range(N_ROUNDS):
            w = 1 << r
            rdma = pltpu.make_async_remote_copy(
                src_ref=gather_ref.at[pl.ds(0, w)],
                dst_ref=gather_ref.at[pl.ds(w, w)],
                send_sem=send_sems.at[r],
                recv_sem=recv_sems.at[r],
                device_id=(partners[r],),
                device_id_type=pl.DeviceIdType.MESH,
            )
            rdma.start()
            rdma.wait()

        v = gather_ref[:, :, :]
        mx = jnp.max(v, axis=(0, 1))
        rows = [mx]
        for _ in range(K - 1):
            mx = jnp.max(
                jnp.where(v < mx[None, None, :], v, NEG_INF), axis=(0, 1)
            )
            rows.append(mx)
        out_t = jnp.stack(rows, axis=0)
        out_ref[:, :] = out_t.T

        @functools.partial(
            pl.run_scoped, second_barrier=pltpu.SemaphoreType.REGULAR
        )
        def _(second_barrier):
            for p in partners:
                pl.semaphore_signal(
                    second_barrier, inc=1,
                    device_id=(p,), device_id_type=pl.DeviceIdType.MESH,
                )
            pl.semaphore_wait(second_barrier, N_ROUNDS)

    return pl.pallas_call(
        body,
        out_shape=jax.ShapeDtypeStruct((m, K), jnp.float32),
        in_specs=[pl.BlockSpec(memory_space=pltpu.VMEM)],
        out_specs=pl.BlockSpec(memory_space=pltpu.VMEM),
        scratch_shapes=[
            pltpu.VMEM((N_DEV, K, m), jnp.float32),
            pltpu.SemaphoreType.DMA((N_ROUNDS,)),
            pltpu.SemaphoreType.DMA((N_ROUNDS,)),
        ],
        compiler_params=pltpu.CompilerParams(collective_id=0),
    )(x)


# device time: 25708 ns/iter; 1.1989x vs baseline; 1.1989x over previous
import functools

import jax
import jax.numpy as jnp
from jax import lax
from jax.experimental import pallas as pl
from jax.experimental.pallas import tpu as pltpu

N_DEV = 8
K = 16
NEG_INF = float("-inf")


def _topk_iter(x, k):
    m = jnp.max(x, axis=1, keepdims=True)
    cols = [m]
    for _ in range(k - 1):
        m = jnp.max(jnp.where(x < m, x, NEG_INF), axis=1, keepdims=True)
        cols.append(m)
    return jnp.concatenate(cols, axis=1)


def kernel(x):
    m, n = x.shape

    def body(x_ref, out_ref, gather_ref, send_sems, recv_sems):
        my_id = lax.axis_index("i")
        peers = [my_id ^ d for d in range(1, N_DEV)]

        barrier = pltpu.get_barrier_semaphore()
        for p in peers:
            pl.semaphore_signal(
                barrier, inc=1,
                device_id=(p,), device_id_type=pl.DeviceIdType.MESH,
            )
        pl.semaphore_wait(barrier, N_DEV - 1)

        gather_ref[0] = _topk_iter(x_ref[:, :], K).T

        rdmas = []
        for d in range(1, N_DEV):
            rdma = pltpu.make_async_remote_copy(
                src_ref=gather_ref.at[0],
                dst_ref=gather_ref.at[d],
                send_sem=send_sems.at[d],
                recv_sem=recv_sems.at[d],
                device_id=(my_id ^ d,),
                device_id_type=pl.DeviceIdType.MESH,
            )
            rdma.start()
            rdmas.append(rdma)
        for rdma in rdmas:
            rdma.wait()

        v = gather_ref[:, :, :]
        mx = jnp.max(v, axis=(0, 1))
        rows = [mx]
        for _ in range(K - 1):
            mx = jnp.max(
                jnp.where(v < mx[None, None, :], v, NEG_INF), axis=(0, 1)
            )
            rows.append(mx)
        out_t = jnp.stack(rows, axis=0)
        out_ref[:, :] = out_t.T

        @functools.partial(
            pl.run_scoped, second_barrier=pltpu.SemaphoreType.REGULAR
        )
        def _(second_barrier):
            for p in peers:
                pl.semaphore_signal(
                    second_barrier, inc=1,
                    device_id=(p,), device_id_type=pl.DeviceIdType.MESH,
                )
            pl.semaphore_wait(second_barrier, N_DEV - 1)

    return pl.pallas_call(
        body,
        out_shape=jax.ShapeDtypeStruct((m, K), jnp.float32),
        in_specs=[pl.BlockSpec(memory_space=pltpu.VMEM)],
        out_specs=pl.BlockSpec(memory_space=pltpu.VMEM),
        scratch_shapes=[
            pltpu.VMEM((N_DEV, K, m), jnp.float32),
            pltpu.SemaphoreType.DMA((N_DEV,)),
            pltpu.SemaphoreType.DMA((N_DEV,)),
        ],
        compiler_params=pltpu.CompilerParams(collective_id=0),
    )(x)
